# baseline (speedup 1.0000x reference)
import functools

import jax
import jax.numpy as jnp
from jax import lax
from jax.experimental import pallas as pl
from jax.experimental.pallas import tpu as pltpu
from jax.experimental.pallas import tpu_sc as plsc

_VOCAB = 100000
_HIDDEN = 1024
_B = 4
_S = 4096
_N = _B * _S

_NC = 2
_NS = 16
_NW = _NC * _NS
_BPW = _N // _NW
_WPB = _S // _BPW
_CHUNK = 8
_NBUF = 8
_DRAIN = 6
_NCHUNK = _BPW // _CHUNK
_NOUTER = _NCHUNK // _NBUF

_mesh = plsc.VectorSubcoreMesh(core_axis_name="c", subcore_axis_name="s")


@functools.partial(
    pl.kernel,
    mesh=_mesh,
    out_type=jax.ShapeDtypeStruct((_B, _S, _HIDDEN), jnp.float32),
    scratch_types=[
        pltpu.VMEM((_BPW,), jnp.int32),
        pltpu.VMEM((_NBUF, _CHUNK, _HIDDEN), jnp.float32),
    ]
    + [pltpu.SemaphoreType.DMA] * (2 * _NBUF),
)
def _gather(idx_hbm, table_hbm, out_hbm, idx_v, rows_v, *sems):
    gsems = sems[:_NBUF]
    osems = sems[_NBUF:]
    wid = lax.axis_index("s") * _NC + lax.axis_index("c")
    bb = wid // _WPB
    cc = (wid % _WPB) * _BPW
    pltpu.sync_copy(idx_hbm.at[bb, pl.ds(cc, _BPW)], idx_v)

    def gather_start(b, i):
        pltpu.async_copy(
            table_hbm.at[idx_v.at[pl.ds(i * _CHUNK, _CHUNK)]],
            rows_v.at[b],
            gsems[b],
        )

    def gather_wait(b):
        pltpu.make_async_copy(
            table_hbm.at[pl.ds(0, _CHUNK)], rows_v.at[b], gsems[b]
        ).wait()

    def out_start(b, i):
        pltpu.async_copy(
            rows_v.at[b],
            out_hbm.at[bb, pl.ds(cc + i * _CHUNK, _CHUNK)],
            osems[b],
        )

    def out_wait(b):
        pltpu.make_async_copy(
            rows_v.at[b], out_hbm.at[0, pl.ds(0, _CHUNK)], osems[b]
        ).wait()

    for b in range(_NBUF):
        gather_start(b, b)

    def body(j, carry):
        for b in range(_NBUF):
            i = j * _NBUF + b
            gather_wait(b)
            out_start(b, i)

            br = (b - _DRAIN) % _NBUF

            @pl.when((i >= _DRAIN) & (i + _NBUF - _DRAIN < _NCHUNK))
            def _():
                out_wait(br)
                gather_start(br, i + _NBUF - _DRAIN)

        return carry

    lax.fori_loop(0, _NOUTER, body, 0)

    for b in range(_NBUF):
        out_wait(b)


def kernel(input_ids, embed_table):
    return _gather(input_ids.astype(jnp.int32), embed_table)


# speedup vs baseline: 1.0730x; 1.0730x over previous
import functools

import jax
import jax.numpy as jnp
from jax import lax
from jax.experimental import pallas as pl
from jax.experimental.pallas import tpu as pltpu
from jax.experimental.pallas import tpu_sc as plsc

_VOCAB = 100000
_HIDDEN = 1024
_B = 4
_S = 4096
_N = _B * _S

_NC = 2
_NS = 16
_NW = _NC * _NS
_BPW = _N // _NW
_WPB = _S // _BPW
_CHUNK = 8
_NBUF = 8
_DRAIN = 4
_NCHUNK = _BPW // _CHUNK
_NOUTER = _NCHUNK // _NBUF

_mesh = plsc.VectorSubcoreMesh(core_axis_name="c", subcore_axis_name="s")


@functools.partial(
    pl.kernel,
    mesh=_mesh,
    out_type=jax.ShapeDtypeStruct((_B, _S, _HIDDEN), jnp.float32),
    scratch_types=[
        pltpu.VMEM((_BPW,), jnp.int32),
        pltpu.VMEM((_NBUF, _CHUNK, _HIDDEN), jnp.float32),
    ]
    + [pltpu.SemaphoreType.DMA] * (2 * _NBUF),
)
def _gather(idx_hbm, table_hbm, out_hbm, idx_v, rows_v, *sems):
    gsems = sems[:_NBUF]
    osems = sems[_NBUF:]
    wid = lax.axis_index("s") * _NC + lax.axis_index("c")
    bb = wid // _WPB
    cc = (wid % _WPB) * _BPW
    pltpu.sync_copy(idx_hbm.at[bb, pl.ds(cc, _BPW)], idx_v)

    def gather_start(b, i):
        pltpu.async_copy(
            table_hbm.at[idx_v.at[pl.ds(i * _CHUNK, _CHUNK)]],
            rows_v.at[b],
            gsems[b],
        )

    def gather_wait(b):
        pltpu.make_async_copy(
            table_hbm.at[pl.ds(0, _CHUNK)], rows_v.at[b], gsems[b]
        ).wait()

    def out_start(b, i):
        pltpu.async_copy(
            rows_v.at[b],
            out_hbm.at[bb, pl.ds(cc + i * _CHUNK, _CHUNK)],
            osems[b],
        )

    def out_wait(b):
        pltpu.make_async_copy(
            rows_v.at[b], out_hbm.at[0, pl.ds(0, _CHUNK)], osems[b]
        ).wait()

    for b in range(_NBUF):
        gather_start(b, b)

    def body(j, carry):
        for b in range(_NBUF):
            i = j * _NBUF + b
            gather_wait(b)
            out_start(b, i)

            br = (b - _DRAIN) % _NBUF

            @pl.when((i >= _DRAIN) & (i + _NBUF - _DRAIN < _NCHUNK))
            def _():
                out_wait(br)
                gather_start(br, i + _NBUF - _DRAIN)

        return carry

    lax.fori_loop(0, _NOUTER, body, 0)

    for b in range(_NBUF):
        out_wait(b)


def kernel(input_ids, embed_table):
    return _gather(input_ids.astype(jnp.int32), embed_table)
